# baseline (speedup 1.0000x reference)
import jax
import jax.numpy as jnp
from jax.experimental import pallas as pl

_K = 10
_ROWS = 256
_H = 16


def _cumsum_lanes(x):
    n = x.shape[-1]
    shift = 1
    while shift < n:
        shifted = jnp.concatenate(
            [jnp.zeros(x.shape[:-1] + (shift,), x.dtype), x[..., :-shift]], axis=-1)
        x = x + shifted
        shift *= 2
    return x


def _tc_kernel(embt_ref, emb_rows_ref, alpha_ref, phys_ref, out_ref):
    embt = embt_ref[0]
    emb_rows = emb_rows_ref[0]
    c = jax.nn.sigmoid(alpha_ref[0, 0])

    a = jax.lax.dot_general(emb_rows, embt, (((1,), (0,)), ((), ())),
                            preferred_element_type=jnp.float32)
    a = jnp.maximum(a, 0.0)

    r, n = a.shape
    ones = jnp.ones((n, 1), dtype=jnp.float32)

    def rowsum(v):
        return jax.lax.dot_general(v, ones, (((1,), (0,)), ((), ())),
                                   preferred_element_type=jnp.float32)

    cur = jnp.full((r, 1), jnp.inf, dtype=jnp.float32)
    thr = jnp.zeros((r, 1), dtype=jnp.float32)
    row_max = jnp.zeros((r, 1), dtype=jnp.float32)
    for k in range(_K):
        lt = a < cur
        n_ge = float(n) - rowsum(jnp.where(lt, 1.0, 0.0))
        d = jnp.max(jnp.where(lt, a, -1.0), axis=1, keepdims=True)
        take = n_ge < float(_K)
        thr = jnp.where(take, d, thr)
        if k == 0:
            row_max = d
        cur = d

    mask_gt = a > thr
    mask_eq = a == thr
    cnt_gt = rowsum(jnp.where(mask_gt, 1.0, 0.0))
    cnt_eq = rowsum(jnp.where(mask_eq, 1.0, 0.0))
    extra = float(_K) - cnt_gt
    has_tie = jnp.max(cnt_eq - extra) > 0.5

    def tie_sel(_):
        rank = _cumsum_lanes(jnp.where(mask_eq, 1.0, 0.0))
        keep = jnp.logical_and(mask_eq, rank <= extra)
        return jnp.where(jnp.logical_or(mask_gt, keep), 1.0, 0.0)

    def fast_sel(_):
        return jnp.where(a >= thr, 1.0, 0.0)

    sel = jax.lax.cond(has_tie, tie_sel, fast_sel, None)

    e = sel * jnp.exp(a - row_max)
    z = rowsum(e)

    phys = phys_ref[...]
    psum = rowsum(phys) + 1e-8
    out_ref[0, :, :] = (c / psum) * phys + ((1.0 - c) / z) * e


def kernel(x, A_physical, W, b, alpha):
    bsz, _, n, _ = x.shape
    state = x[:, -1, :, :]
    emb = jnp.tanh(state @ W + b)
    embt = jnp.swapaxes(emb, 1, 2)
    alpha2 = jnp.asarray(alpha, jnp.float32).reshape(1, 1)
    grid = (n // _ROWS, bsz)
    return pl.pallas_call(
        _tc_kernel,
        grid=grid,
        in_specs=[
            pl.BlockSpec((1, _H, n), lambda i, bb: (bb, 0, 0)),
            pl.BlockSpec((1, _ROWS, _H), lambda i, bb: (bb, i, 0)),
            pl.BlockSpec((1, 1), lambda i, bb: (0, 0)),
            pl.BlockSpec((_ROWS, n), lambda i, bb: (i, 0)),
        ],
        out_specs=pl.BlockSpec((1, _ROWS, n), lambda i, bb: (bb, i, 0)),
        out_shape=jax.ShapeDtypeStruct((bsz, n, n), jnp.float32),
    )(embt, emb, alpha2, A_physical)


# speedup vs baseline: 1.1022x; 1.1022x over previous
import jax
import jax.numpy as jnp
from jax.experimental import pallas as pl

_K = 10
_ROWS = 256
_H = 16


def _cumsum_lanes(x):
    n = x.shape[-1]
    shift = 1
    while shift < n:
        shifted = jnp.concatenate(
            [jnp.zeros(x.shape[:-1] + (shift,), x.dtype), x[..., :-shift]], axis=-1)
        x = x + shifted
        shift *= 2
    return x


def _tc_kernel(embt_ref, emb_rows_ref, alpha_ref, phys_ref, out_ref):
    embt = embt_ref[0]
    emb_rows = emb_rows_ref[0]
    c = jax.nn.sigmoid(alpha_ref[0, 0])

    a = jax.lax.dot_general(emb_rows, embt, (((1,), (0,)), ((), ())),
                            preferred_element_type=jnp.float32)
    a = jnp.maximum(a, 0.0)

    r, n = a.shape
    ones = jnp.ones((n, 1), dtype=jnp.float32)

    def rowsum(v):
        return jax.lax.dot_general(v, ones, (((1,), (0,)), ((), ())),
                                   preferred_element_type=jnp.float32)

    cur = jnp.full((r, 1), jnp.inf, dtype=jnp.float32)
    thr = jnp.zeros((r, 1), dtype=jnp.float32)
    row_max = jnp.zeros((r, 1), dtype=jnp.float32)
    for k in range(_K):
        lt = a < cur
        n_ge = float(n) - rowsum(jnp.where(lt, 1.0, 0.0))
        d = jnp.max(jnp.where(lt, a, -1.0), axis=1, keepdims=True)
        take = n_ge < float(_K)
        thr = jnp.where(take, d, thr)
        if k == 0:
            row_max = d
        cur = d

    mask_gt = a > thr
    mask_eq = a == thr
    cnt_gt = rowsum(jnp.where(mask_gt, 1.0, 0.0))
    cnt_eq = rowsum(jnp.where(mask_eq, 1.0, 0.0))
    extra = float(_K) - cnt_gt
    has_tie = jnp.max(cnt_eq - extra) > 0.5

    phys = phys_ref[...]
    psum = jnp.sum(phys, axis=1, keepdims=True) + 1e-8
    ex = jnp.exp(a - row_max)

    def emit(sel):
        e = sel * ex
        z = jnp.sum(e, axis=1, keepdims=True)
        out_ref[0, :, :] = (c / psum) * phys + ((1.0 - c) / z) * e

    @pl.when(jnp.logical_not(has_tie))
    def _fast():
        emit(jnp.where(a >= thr, 1.0, 0.0))

    @pl.when(has_tie)
    def _tie():
        rank = _cumsum_lanes(jnp.where(mask_eq, 1.0, 0.0))
        keep = jnp.logical_and(mask_eq, rank <= extra)
        emit(jnp.where(jnp.logical_or(mask_gt, keep), 1.0, 0.0))


def kernel(x, A_physical, W, b, alpha):
    bsz, _, n, _ = x.shape
    state = x[:, -1, :, :]
    emb = jnp.tanh(state @ W + b)
    embt = jnp.swapaxes(emb, 1, 2)
    alpha2 = jnp.asarray(alpha, jnp.float32).reshape(1, 1)
    grid = (n // _ROWS, bsz)
    return pl.pallas_call(
        _tc_kernel,
        grid=grid,
        in_specs=[
            pl.BlockSpec((1, _H, n), lambda i, bb: (bb, 0, 0)),
            pl.BlockSpec((1, _ROWS, _H), lambda i, bb: (bb, i, 0)),
            pl.BlockSpec((1, 1), lambda i, bb: (0, 0)),
            pl.BlockSpec((_ROWS, n), lambda i, bb: (i, 0)),
        ],
        out_specs=pl.BlockSpec((1, _ROWS, n), lambda i, bb: (bb, i, 0)),
        out_shape=jax.ShapeDtypeStruct((bsz, n, n), jnp.float32),
    )(embt, emb, alpha2, A_physical)
